# Initial kernel scaffold; baseline (speedup 1.0000x reference)
import jax
import jax.numpy as jnp
from jax.experimental import pallas as pl


def kernel(inputs, E0, E1, E2, E3):
    raise NotImplementedError("write your pallas kernel here")


# speedup vs baseline: 2.1277x; 2.1277x over previous
import jax
import jax.numpy as jnp
from jax.experimental import pallas as pl

NLEV = 4
NCODE = 1024
DIM = 64
ROWS = 16 * 576
BLK = 512


def _body(x_ref, e_ref, q_ref, idx_ref, loss_ref):
    x = x_ref[:]
    res = x
    total_q = jnp.zeros_like(x)
    loss = jnp.float32(0.0)
    idxs = []
    iota = jax.lax.broadcasted_iota(jnp.int32, (BLK, NCODE), 1)
    for l in range(NLEV):
        E = e_ref[l * NCODE:(l + 1) * NCODE, :]
        e_sq = jnp.sum(E * E, axis=1)
        row_sq = jnp.sum(res * res, axis=1, keepdims=True)
        mm = jax.lax.dot_general(res, E, (((1,), (1,)), ((), ())),
                                 preferred_element_type=jnp.float32)
        d = (row_sq + e_sq[None, :]) - 2.0 * mm
        min_d = jnp.min(d, axis=1, keepdims=True)
        idx = jnp.min(jnp.where(d == min_d, iota, NCODE), axis=1)
        onehot = (iota == idx[:, None]).astype(jnp.float32)
        q = jax.lax.dot_general(onehot, E, (((1,), (0,)), ((), ())),
                                preferred_element_type=jnp.float32)
        idxs.append(idx)
        total_q = total_q + q
        res = res - q
        loss = loss + jnp.sum(res * res)
    q_ref[:] = x + (total_q - x)
    idx_ref[:] = jnp.stack(idxs, axis=0)

    @pl.when(pl.program_id(0) == 0)
    def _():
        loss_ref[:, :] = jnp.zeros((1, 1), jnp.float32)

    loss_ref[:, :] = loss_ref[:, :] + jnp.broadcast_to(loss, (1, 1))


def kernel(inputs, E0, E1, E2, E3):
    x = inputs.reshape(ROWS, DIM)
    ecat = jnp.concatenate([E0, E1, E2, E3], axis=0)
    grid = ROWS // BLK
    q, idx, loss = pl.pallas_call(
        _body,
        grid=(grid,),
        in_specs=[
            pl.BlockSpec((BLK, DIM), lambda i: (i, 0)),
            pl.BlockSpec((NLEV * NCODE, DIM), lambda i: (0, 0)),
        ],
        out_specs=[
            pl.BlockSpec((BLK, DIM), lambda i: (i, 0)),
            pl.BlockSpec((NLEV, BLK), lambda i: (0, i)),
            pl.BlockSpec((1, 1), lambda i: (0, 0)),
        ],
        out_shape=[
            jax.ShapeDtypeStruct((ROWS, DIM), jnp.float32),
            jax.ShapeDtypeStruct((NLEV, ROWS), jnp.int32),
            jax.ShapeDtypeStruct((1, 1), jnp.float32),
        ],
    )(x, ecat)
    quantized_ste = q.reshape(inputs.shape)
    scale = jnp.float32(1.0 / (ROWS * DIM * NLEV))
    loss_out = loss[0, 0] * scale
    indices = idx.reshape(NLEV, *inputs.shape[:-1])
    return (quantized_ste, loss_out, loss_out, indices)


# speedup vs baseline: 2.3593x; 1.1089x over previous
import jax
import jax.numpy as jnp
from jax.experimental import pallas as pl
from jax.experimental.pallas import tpu as pltpu

NLEV = 4
NCODE = 1024
DIM = 64
ROWS = 16 * 576
BLK = 1024


def _body(x_ref, e_ref, q_ref, idx_ref, loss_ref, es_ref):
    @pl.when(pl.program_id(0) == 0)
    def _():
        es = [jnp.sum(e_ref[l * NCODE:(l + 1) * NCODE, :] ** 2, axis=1)
              for l in range(NLEV)]
        es_ref[:, :] = jnp.stack(es, axis=0)
        loss_ref[:, :] = jnp.zeros((1, 1), jnp.float32)

    x = x_ref[:]
    res = x
    total_q = jnp.zeros_like(x)
    loss = jnp.float32(0.0)
    idxs = []
    iota = jax.lax.broadcasted_iota(jnp.int32, (BLK, NCODE), 1)
    for l in range(NLEV):
        E = e_ref[l * NCODE:(l + 1) * NCODE, :]
        e_sq = es_ref[l:l + 1, :]
        row_sq = jnp.sum(res * res, axis=1, keepdims=True)
        mm = jax.lax.dot_general(res, E, (((1,), (1,)), ((), ())),
                                 preferred_element_type=jnp.float32)
        d = (row_sq + e_sq) - 2.0 * mm
        min_d = jnp.min(d, axis=1, keepdims=True)
        idx = jnp.min(jnp.where(d == min_d, iota, NCODE), axis=1)
        onehot = (iota == idx[:, None]).astype(jnp.float32)
        q = jax.lax.dot_general(onehot, E, (((1,), (0,)), ((), ())),
                                preferred_element_type=jnp.float32)
        idxs.append(idx)
        total_q = total_q + q
        res = res - q
        loss = loss + jnp.sum(res * res)
    q_ref[:] = x + (total_q - x)
    idx_ref[:] = jnp.stack(idxs, axis=0)
    loss_ref[:, :] = loss_ref[:, :] + jnp.broadcast_to(loss, (1, 1))


def kernel(inputs, E0, E1, E2, E3):
    x = inputs.reshape(ROWS, DIM)
    ecat = jnp.concatenate([E0, E1, E2, E3], axis=0)
    grid = ROWS // BLK
    q, idx, loss = pl.pallas_call(
        _body,
        grid=(grid,),
        in_specs=[
            pl.BlockSpec((BLK, DIM), lambda i: (i, 0)),
            pl.BlockSpec((NLEV * NCODE, DIM), lambda i: (0, 0)),
        ],
        out_specs=[
            pl.BlockSpec((BLK, DIM), lambda i: (i, 0)),
            pl.BlockSpec((NLEV, BLK), lambda i: (0, i)),
            pl.BlockSpec((1, 1), lambda i: (0, 0)),
        ],
        out_shape=[
            jax.ShapeDtypeStruct((ROWS, DIM), jnp.float32),
            jax.ShapeDtypeStruct((NLEV, ROWS), jnp.int32),
            jax.ShapeDtypeStruct((1, 1), jnp.float32),
        ],
        scratch_shapes=[pltpu.VMEM((NLEV, NCODE), jnp.float32)],
    )(x, ecat)
    quantized_ste = q.reshape(inputs.shape)
    scale = jnp.float32(1.0 / (ROWS * DIM * NLEV))
    loss_out = loss[0, 0] * scale
    indices = idx.reshape(NLEV, *inputs.shape[:-1])
    return (quantized_ste, loss_out, loss_out, indices)
